# baseline (speedup 1.0000x reference)
import functools

import jax
import jax.numpy as jnp
from jax import lax
from jax.experimental import pallas as pl
from jax.experimental.pallas import tpu as pltpu
from jax.experimental.pallas import tpu_sc as plsc

_B = 4096
_S = 50
_D = 64
_V = 1000000
_NC = 2
_NS = 16
_CPC = _D // _NC
_BH = _B // 128
_MAXNS = 4


def _sc_body(tokt_hbm, lbl_hbm, tablet_hbm, st_hbm, pt_hbm,
             hyb_out, tok_out,
             tok_idx, lbl_v, st_v, pt_v, svec, g1, tbuf, hbuf,
             crow, sem_g, sem_to, sem_ho):
    core = lax.axis_index("c")
    sid = lax.axis_index("s")
    sa = lax.div(sid * _S, _NS)
    sb = lax.div((sid + 1) * _S, _NS)
    ns = sb - sa

    pltpu.sync_copy(st_hbm, st_v)
    pltpu.sync_copy(pt_hbm, pt_v)
    pltpu.sync_copy(lbl_hbm, lbl_v)

    @pl.loop(0, ns)
    def _stage_tok(i):
        pltpu.sync_copy(tokt_hbm.at[sa + i], tok_idx.at[i])

    @pl.loop(0, _CPC)
    def _comp(j):
        c = core * _CPC + j
        @pl.when(sid == 0)
        def _stage():
            pltpu.sync_copy(tablet_hbm.at[c], crow)

        plsc.subcore_barrier()

        cbase = c * 4

        @pl.loop(0, _BH)
        def _mkstyle(r):
            for k in range(8):
                sl = pl.ds(k * 16, 16)
                idx = jnp.full((16,), cbase, jnp.int32) + lbl_v[r, sl]
                svec[r, sl] = plsc.load_gather(st_v, [idx])

        @pl.loop(0, ns)
        def _plane(i):
            s = sa + i
            pltpu.async_copy(crow.at[tok_idx.at[i]], g1, sem_g).wait()
            @pl.when(jnp.logical_or(j > 0, i > 0))
            def _drain():
                pltpu.make_async_copy(
                    tbuf, tok_out.at[0, 0, :, 0, :], sem_to).wait()
                pltpu.make_async_copy(
                    hbuf, hyb_out.at[0, 0, :, 0, :], sem_ho).wait()

            pidx = jnp.full((16,), c * _S + s, jnp.int32)
            pspl = plsc.load_gather(pt_v, [pidx])

            @pl.loop(0, _BH)
            def _add(r):
                for k in range(8):
                    sl = pl.ds(k * 16, 16)
                    g = g1[pl.ds(r * 128 + k * 16, 16)]
                    tbuf[r, sl] = g
                    hbuf[r, sl] = g + svec[r, sl] + pspl

            chi = lax.div(c, 8)
            clo = lax.rem(c, 8)
            pltpu.async_copy(tbuf, tok_out.at[s, chi, :, clo, :], sem_to)
            pltpu.async_copy(hbuf, hyb_out.at[s, chi, :, clo, :], sem_ho)

        plsc.subcore_barrier()

    pltpu.make_async_copy(tbuf, tok_out.at[0, 0, :, 0, :], sem_to).wait()
    pltpu.make_async_copy(hbuf, hyb_out.at[0, 0, :, 0, :], sem_ho).wait()


@jax.jit
def _sc_call(tokt, lbl2, tablet, st_flat, pt_flat):
    mesh = plsc.VectorSubcoreMesh(core_axis_name="c", subcore_axis_name="s")
    run = pl.kernel(
        _sc_body,
        out_type=(
            jax.ShapeDtypeStruct((_S, 8, _BH, 8, 128), jnp.float32),
            jax.ShapeDtypeStruct((_S, 8, _BH, 8, 128), jnp.float32),
        ),
        mesh=mesh,
        scratch_types=[
            pltpu.VMEM((_MAXNS, _B), jnp.int32),
            pltpu.VMEM((_BH, 128), jnp.int32),
            pltpu.VMEM((4 * _D,), jnp.float32),
            pltpu.VMEM((_D * _S,), jnp.float32),
            pltpu.VMEM((_BH, 128), jnp.float32),
            pltpu.VMEM((_B,), jnp.float32),
            pltpu.VMEM((_BH, 128), jnp.float32),
            pltpu.VMEM((_BH, 128), jnp.float32),
            pltpu.VMEM_SHARED((_V,), jnp.float32),
            pltpu.SemaphoreType.DMA,
            pltpu.SemaphoreType.DMA,
            pltpu.SemaphoreType.DMA,
        ],
        compiler_params=pltpu.CompilerParams(
            use_tc_tiling_on_sc=False, needs_layout_passes=False),
    )
    return run(tokt, lbl2, tablet, st_flat, pt_flat)


def kernel(tokens, labels, token_table, style_table, posit_table):
    seq = tokens.shape[1]
    tokt = tokens.T.astype(jnp.int32)
    lbl2 = labels.reshape(_BH, 128).astype(jnp.int32)
    tablet = token_table.T
    st_flat = style_table.T.reshape(-1)
    pt_flat = posit_table[:seq].T.reshape(-1)
    hyb5, tok5 = _sc_call(tokt, lbl2, tablet, st_flat, pt_flat)
    hyb = hyb5.transpose(2, 4, 0, 1, 3).reshape(_B, seq, _D)
    tok = tok5.transpose(2, 4, 0, 1, 3).reshape(_B, seq, _D)
    return hyb, tok


# speedup vs baseline: 12.4095x; 12.4095x over previous
import functools

import jax
import jax.numpy as jnp
from jax import lax
from jax.experimental import pallas as pl
from jax.experimental.pallas import tpu as pltpu
from jax.experimental.pallas import tpu_sc as plsc

_B = 4096
_S = 50
_D = 64
_V = 1000000
_NC = 2
_NS = 16
_CPC = _D // _NC
_BH = _B // 128
_MAXNS = 4


def _sc_body(tokt_hbm, lbl_hbm, tablet_hbm, st_hbm, pt_hbm,
             hyb_out, tok_out,
             idx0, idx1, idx2, idx3, lbl_v, st_v, pt_v, svec, g1, tbuf, hbuf,
             crow, sem_g, sem_to, sem_ho):
    core = lax.axis_index("c")
    sid = lax.axis_index("s")
    idx_bufs = (idx0, idx1, idx2, idx3)

    pltpu.sync_copy(st_hbm, st_v)
    pltpu.sync_copy(pt_hbm, pt_v)
    pltpu.sync_copy(lbl_hbm, lbl_v)

    for k in range(_MAXNS):
        if k < 3:
            pltpu.sync_copy(tokt_hbm.at[sid + 16 * k], idx_bufs[k])
        else:
            @pl.when(sid + 16 * k < _S)
            def _stage_last():
                pltpu.sync_copy(tokt_hbm.at[sid + 16 * k], idx_bufs[k])

    @pl.loop(0, _CPC)
    def _comp(j):
        c = core * _CPC + j
        @pl.when(sid == 0)
        def _stage():
            pltpu.sync_copy(tablet_hbm.at[c], crow)

        plsc.subcore_barrier()

        cbase = c * 4

        @pl.loop(0, _BH)
        def _mkstyle(r):
            for k in range(8):
                sl = pl.ds(k * 16, 16)
                idx = jnp.full((16,), cbase, jnp.int32) + lbl_v[r, sl]
                svec[r, sl] = plsc.load_gather(st_v, [idx])

        chi = lax.div(c, 8)
        clo = lax.rem(c, 8)

        def _do_plane(k):
            s = sid + 16 * k
            pltpu.async_copy(crow.at[idx_bufs[k]], g1, sem_g).wait()
            @pl.when(jnp.logical_or(j > 0, k > 0))
            def _drain():
                pltpu.make_async_copy(
                    tbuf, tok_out.at[0, 0, :, 0, :], sem_to).wait()
                pltpu.make_async_copy(
                    hbuf, hyb_out.at[0, 0, :, 0, :], sem_ho).wait()

            pidx = jnp.full((16,), c * _S + s, jnp.int32)
            pspl = plsc.load_gather(pt_v, [pidx])

            @pl.loop(0, _BH)
            def _add(r):
                for kk in range(8):
                    sl = pl.ds(kk * 16, 16)
                    g = g1[pl.ds(r * 128 + kk * 16, 16)]
                    tbuf[r, sl] = g
                    hbuf[r, sl] = g + svec[r, sl] + pspl

            pltpu.async_copy(tbuf, tok_out.at[s, chi, :, clo, :], sem_to)
            pltpu.async_copy(hbuf, hyb_out.at[s, chi, :, clo, :], sem_ho)

        for k in range(_MAXNS):
            if k < 3:
                _do_plane(k)
            else:
                @pl.when(sid + 16 * k < _S)
                def _last_plane():
                    _do_plane(k)

        plsc.subcore_barrier()

    pltpu.make_async_copy(tbuf, tok_out.at[0, 0, :, 0, :], sem_to).wait()
    pltpu.make_async_copy(hbuf, hyb_out.at[0, 0, :, 0, :], sem_ho).wait()


@jax.jit
def _sc_call(tokt, lbl2, tablet, st_flat, pt_flat):
    mesh = plsc.VectorSubcoreMesh(core_axis_name="c", subcore_axis_name="s")
    run = pl.kernel(
        _sc_body,
        out_type=(
            jax.ShapeDtypeStruct((_S, 8, _BH, 8, 128), jnp.float32),
            jax.ShapeDtypeStruct((_S, 8, _BH, 8, 128), jnp.float32),
        ),
        mesh=mesh,
        scratch_types=[
            pltpu.VMEM((_B,), jnp.int32),
            pltpu.VMEM((_B,), jnp.int32),
            pltpu.VMEM((_B,), jnp.int32),
            pltpu.VMEM((_B,), jnp.int32),
            pltpu.VMEM((_BH, 128), jnp.int32),
            pltpu.VMEM((4 * _D,), jnp.float32),
            pltpu.VMEM((_D * _S,), jnp.float32),
            pltpu.VMEM((_BH, 128), jnp.float32),
            pltpu.VMEM((_B,), jnp.float32),
            pltpu.VMEM((_BH, 128), jnp.float32),
            pltpu.VMEM((_BH, 128), jnp.float32),
            pltpu.VMEM_SHARED((_V,), jnp.float32),
            pltpu.SemaphoreType.DMA,
            pltpu.SemaphoreType.DMA,
            pltpu.SemaphoreType.DMA,
        ],
        compiler_params=pltpu.CompilerParams(
            use_tc_tiling_on_sc=True, needs_layout_passes=False),
    )
    return run(tokt, lbl2, tablet, st_flat, pt_flat)


def kernel(tokens, labels, token_table, style_table, posit_table):
    seq = tokens.shape[1]
    tokt = tokens.T.astype(jnp.int32)
    lbl2 = labels.reshape(_BH, 128).astype(jnp.int32)
    tablet = token_table.T
    st_flat = style_table.T.reshape(-1)
    pt_flat = posit_table[:seq].T.reshape(-1)
    hyb5, tok5 = _sc_call(tokt, lbl2, tablet, st_flat, pt_flat)
    hyb = hyb5.transpose(2, 4, 0, 1, 3).reshape(_B, seq, _D)
    tok = tok5.transpose(2, 4, 0, 1, 3).reshape(_B, seq, _D)
    return hyb, tok


# speedup vs baseline: 14.1788x; 1.1426x over previous
import functools

import jax
import jax.numpy as jnp
from jax import lax
from jax.experimental import pallas as pl
from jax.experimental.pallas import tpu as pltpu
from jax.experimental.pallas import tpu_sc as plsc

_B = 4096
_S = 50
_D = 64
_V = 1000000
_NC = 2
_NS = 16
_CPC = _D // _NC
_BH = _B // 128
_MAXNS = 4


_STG = 125056


def _sc_body(tokt_hbm, lbl_hbm, tablet_hbm, st_hbm, pt_hbm,
             hyb_out, tok_out,
             idx0, idx1, idx2, idx3, lbl_v, st_v, pt_v, svec,
             g0, g1, g2, g3, tbuf, hbuf,
             crow, sem_st, sg0, sg1, sg2, sg3, sem_to, sem_ho):
    core = lax.axis_index("c")
    sid = lax.axis_index("s")
    idx_bufs = (idx0, idx1, idx2, idx3)
    g_bufs = (g0, g1, g2, g3)
    sem_g = (sg0, sg1, sg2, sg3)

    pltpu.sync_copy(st_hbm, st_v)
    pltpu.sync_copy(pt_hbm, pt_v)
    pltpu.sync_copy(lbl_hbm, lbl_v)

    for k in range(_MAXNS):
        if k < 3:
            pltpu.sync_copy(tokt_hbm.at[sid + 16 * k], idx_bufs[k])
        else:
            @pl.when(sid + 16 * k < _S)
            def _stage_last():
                pltpu.sync_copy(tokt_hbm.at[sid + 16 * k], idx_bufs[k])

    @pl.loop(0, _CPC)
    def _comp(j):
        c = core * _CPC + j
        @pl.when(sid == 0)
        def _stage():
            pltpu.async_copy(tablet_hbm.at[c], crow, sem_st)

        cbase = c * 4

        @pl.loop(0, _BH)
        def _mkstyle(r):
            for k in range(8):
                sl = pl.ds(k * 16, 16)
                idx = jnp.full((16,), cbase, jnp.int32) + lbl_v[r, sl]
                svec[r, sl] = plsc.load_gather(st_v, [idx])

        @pl.when(sid == 0)
        def _stage_wait():
            pltpu.make_async_copy(tablet_hbm.at[c], crow, sem_st).wait()

        plsc.subcore_barrier()

        for k in range(_MAXNS):
            if k < 3:
                pltpu.async_copy(crow.at[idx_bufs[k]], g_bufs[k], sem_g[k])
            else:
                @pl.when(sid + 16 * k < _S)
                def _fire_last():
                    pltpu.async_copy(crow.at[idx_bufs[k]], g_bufs[k],
                                     sem_g[k])

        chi = lax.div(c, 8)
        clo = lax.rem(c, 8)

        def _do_plane(k):
            s = sid + 16 * k
            gk = g_bufs[k]
            pltpu.make_async_copy(crow.at[idx_bufs[k]], gk, sem_g[k]).wait()
            @pl.when(jnp.logical_or(j > 0, k > 0))
            def _drain():
                pltpu.make_async_copy(
                    tbuf, tok_out.at[0, 0, :, 0, :], sem_to).wait()
                pltpu.make_async_copy(
                    hbuf, hyb_out.at[0, 0, :, 0, :], sem_ho).wait()

            pidx = jnp.full((16,), c * _S + s, jnp.int32)
            pspl = plsc.load_gather(pt_v, [pidx])

            @pl.loop(0, _BH)
            def _add(r):
                for kk in range(8):
                    sl = pl.ds(kk * 16, 16)
                    g = gk[pl.ds(r * 128 + kk * 16, 16)]
                    tbuf[r, sl] = g
                    hbuf[r, sl] = g + svec[r, sl] + pspl

            pltpu.async_copy(tbuf, tok_out.at[s, chi, :, clo, :], sem_to)
            pltpu.async_copy(hbuf, hyb_out.at[s, chi, :, clo, :], sem_ho)

        for k in range(_MAXNS):
            if k < 3:
                _do_plane(k)
            else:
                @pl.when(sid + 16 * k < _S)
                def _last_plane():
                    _do_plane(k)

        plsc.subcore_barrier()

    pltpu.make_async_copy(tbuf, tok_out.at[0, 0, :, 0, :], sem_to).wait()
    pltpu.make_async_copy(hbuf, hyb_out.at[0, 0, :, 0, :], sem_ho).wait()


@jax.jit
def _sc_call(tokt, lbl2, tablet, st_flat, pt_flat):
    mesh = plsc.VectorSubcoreMesh(core_axis_name="c", subcore_axis_name="s")
    run = pl.kernel(
        _sc_body,
        out_type=(
            jax.ShapeDtypeStruct((_S, 8, _BH, 8, 128), jnp.float32),
            jax.ShapeDtypeStruct((_S, 8, _BH, 8, 128), jnp.float32),
        ),
        mesh=mesh,
        scratch_types=[
            pltpu.VMEM((_B,), jnp.int32),
            pltpu.VMEM((_B,), jnp.int32),
            pltpu.VMEM((_B,), jnp.int32),
            pltpu.VMEM((_B,), jnp.int32),
            pltpu.VMEM((_BH, 128), jnp.int32),
            pltpu.VMEM((4 * _D,), jnp.float32),
            pltpu.VMEM((_D * _S,), jnp.float32),
            pltpu.VMEM((_BH, 128), jnp.float32),
            pltpu.VMEM((_B,), jnp.float32),
            pltpu.VMEM((_B,), jnp.float32),
            pltpu.VMEM((_B,), jnp.float32),
            pltpu.VMEM((_B,), jnp.float32),
            pltpu.VMEM((_BH, 128), jnp.float32),
            pltpu.VMEM((_BH, 128), jnp.float32),
            pltpu.VMEM_SHARED((_V,), jnp.float32),
        ] + [pltpu.SemaphoreType.DMA] * 7,
        compiler_params=pltpu.CompilerParams(
            use_tc_tiling_on_sc=True, needs_layout_passes=False),
    )
    return run(tokt, lbl2, tablet, st_flat, pt_flat)


def kernel(tokens, labels, token_table, style_table, posit_table):
    seq = tokens.shape[1]
    tokt = tokens.T.astype(jnp.int32)
    lbl2 = labels.reshape(_BH, 128).astype(jnp.int32)
    tablet = token_table.T
    st_flat = style_table.T.reshape(-1)
    pt_flat = posit_table[:seq].T.reshape(-1)
    hyb5, tok5 = _sc_call(tokt, lbl2, tablet, st_flat, pt_flat)
    hyb = hyb5.transpose(2, 4, 0, 1, 3).reshape(_B, seq, _D)
    tok = tok5.transpose(2, 4, 0, 1, 3).reshape(_B, seq, _D)
    return hyb, tok


# speedup vs baseline: 15.2286x; 1.0740x over previous
import functools

import jax
import jax.numpy as jnp
from jax import lax
from jax.experimental import pallas as pl
from jax.experimental.pallas import tpu as pltpu
from jax.experimental.pallas import tpu_sc as plsc

_B = 4096
_S = 50
_D = 64
_V = 1000000
_NC = 2
_NS = 16
_CPC = _D // _NC
_MAXNS = 4


def _sc_body(tokt_hbm, lbl_hbm, tablet_hbm, st_hbm, pt_hbm,
             hyb_out, tok_out,
             idx0, idx1, idx2, idx3, lbl_v, st_v, pt_v, svec,
             g0, g1, g2, g3, hb0, hb1,
             crow, sem_st, sg0, sg1, sg2, sg3, sem_to, sem_ho):
    core = lax.axis_index("c")
    sid = lax.axis_index("s")
    idx_bufs = (idx0, idx1, idx2, idx3)
    g_bufs = (g0, g1, g2, g3)
    sem_g = (sg0, sg1, sg2, sg3)
    h_bufs = (hb0, hb1)

    pltpu.sync_copy(st_hbm, st_v)
    pltpu.sync_copy(pt_hbm, pt_v)
    pltpu.sync_copy(lbl_hbm, lbl_v)

    def _if_valid(k, fn):
        if k < 3:
            fn()
        else:
            @pl.when(sid + 16 * k < _S)
            def _guarded():
                fn()

    for k in range(_MAXNS):
        _if_valid(k, lambda k=k: pltpu.sync_copy(
            tokt_hbm.at[sid + 16 * k], idx_bufs[k]))

    @pl.when(sid == 0)
    def _stage0():
        pltpu.async_copy(tablet_hbm.at[core * _CPC], crow, sem_st)

    @pl.loop(0, _CPC)
    def _comp(j):
        c = core * _CPC + j
        cbase = c * 4

        @pl.loop(0, _B // 16)
        def _mkstyle(v):
            sl = pl.ds(v * 16, 16)
            idx = jnp.full((16,), cbase, jnp.int32) + lbl_v[sl]
            svec[sl] = plsc.load_gather(st_v, [idx])

        @pl.when(sid == 0)
        def _stage_wait():
            pltpu.make_async_copy(tablet_hbm.at[c], crow, sem_st).wait()

        plsc.subcore_barrier()

        for k in range(_MAXNS):
            def _fire(k=k):
                @pl.when(j > 0)
                def _drain_tok():
                    pltpu.make_async_copy(
                        g_bufs[k], tok_out.at[0, 0, :], sem_to).wait()
                pltpu.async_copy(crow.at[idx_bufs[k]], g_bufs[k], sem_g[k])
            _if_valid(k, _fire)

        for k in range(_MAXNS):
            _if_valid(k, lambda k=k: pltpu.make_async_copy(
                crow.at[idx_bufs[k]], g_bufs[k], sem_g[k]).wait())

        plsc.subcore_barrier()

        @pl.when(jnp.logical_and(sid == 0, j + 1 < _CPC))
        def _stage_next():
            pltpu.async_copy(tablet_hbm.at[c + 1], crow, sem_st)

        def _do_plane(k):
            s = sid + 16 * k
            gk = g_bufs[k]
            hb = h_bufs[k % 2]
            pltpu.async_copy(gk, tok_out.at[s, c, :], sem_to)

            if k == 0:
                prev = j > 0
            elif k == 1:
                prev = jnp.logical_and(j > 0, sid + 48 < _S)
            else:
                prev = True
            @pl.when(prev)
            def _drain_hyb():
                pltpu.make_async_copy(
                    hb, hyb_out.at[0, 0, :], sem_ho).wait()

            pidx = jnp.full((16,), c * _S + s, jnp.int32)
            pspl = plsc.load_gather(pt_v, [pidx])

            @pl.loop(0, _B // 16)
            def _add(v):
                sl = pl.ds(v * 16, 16)
                hb[sl] = gk[sl] + svec[sl] + pspl

            pltpu.async_copy(hb, hyb_out.at[s, c, :], sem_ho)

        for k in range(_MAXNS):
            _if_valid(k, lambda k=k: _do_plane(k))

    for k in range(_MAXNS):
        _if_valid(k, lambda k=k: pltpu.make_async_copy(
            g_bufs[k], tok_out.at[0, 0, :], sem_to).wait())
    for b in range(2):
        pltpu.make_async_copy(h_bufs[b], hyb_out.at[0, 0, :], sem_ho).wait()


@jax.jit
def _sc_call(tokt, lbl, tablet, st_flat, pt_flat):
    mesh = plsc.VectorSubcoreMesh(core_axis_name="c", subcore_axis_name="s")
    run = pl.kernel(
        _sc_body,
        out_type=(
            jax.ShapeDtypeStruct((_S, _D, _B), jnp.float32),
            jax.ShapeDtypeStruct((_S, _D, _B), jnp.float32),
        ),
        mesh=mesh,
        scratch_types=[
            pltpu.VMEM((_B,), jnp.int32),
            pltpu.VMEM((_B,), jnp.int32),
            pltpu.VMEM((_B,), jnp.int32),
            pltpu.VMEM((_B,), jnp.int32),
            pltpu.VMEM((_B,), jnp.int32),
            pltpu.VMEM((4 * _D,), jnp.float32),
            pltpu.VMEM((_D * _S,), jnp.float32),
            pltpu.VMEM((_B,), jnp.float32),
            pltpu.VMEM((_B,), jnp.float32),
            pltpu.VMEM((_B,), jnp.float32),
            pltpu.VMEM((_B,), jnp.float32),
            pltpu.VMEM((_B,), jnp.float32),
            pltpu.VMEM((_B,), jnp.float32),
            pltpu.VMEM((_B,), jnp.float32),
            pltpu.VMEM_SHARED((_V,), jnp.float32),
        ] + [pltpu.SemaphoreType.DMA] * 7,
        compiler_params=pltpu.CompilerParams(
            use_tc_tiling_on_sc=True, needs_layout_passes=False),
    )
    return run(tokt, lbl, tablet, st_flat, pt_flat)


def kernel(tokens, labels, token_table, style_table, posit_table):
    seq = tokens.shape[1]
    tokt = tokens.T.astype(jnp.int32)
    lbl = labels.astype(jnp.int32)
    tablet = token_table.T
    st_flat = style_table.T.reshape(-1)
    pt_flat = posit_table[:seq].T.reshape(-1)
    hyb3, tok3 = _sc_call(tokt, lbl, tablet, st_flat, pt_flat)
    return hyb3.transpose(2, 0, 1), tok3.transpose(2, 0, 1)
